# baseline (speedup 1.0000x reference)
import functools

import jax
import jax.numpy as jnp
from jax import lax
from jax.experimental import pallas as pl
from jax.experimental.pallas import tpu as pltpu
from jax.experimental.pallas import tpu_sc as plsc

_L = 16


@functools.lru_cache(maxsize=None)
def _build(n_pix: int):
    NW = 32
    per_w = n_pix // NW
    assert per_w * NW == n_pix
    P = 4992
    CH = per_w // P
    assert CH * P == per_w
    GRPS = P // _L

    mesh = plsc.VectorSubcoreMesh(core_axis_name="c", subcore_axis_name="s")

    @functools.partial(
        pl.kernel,
        mesh=mesh,
        out_type=(
            jax.ShapeDtypeStruct((n_pix * 4,), jnp.float32),
            jax.ShapeDtypeStruct((n_pix * 4,), jnp.float32),
        ),
        scratch_types=[
            pltpu.VMEM((3 * P,), jnp.float32),
            pltpu.VMEM((4 * P,), jnp.float32),
            pltpu.VMEM((4 * P,), jnp.float32),
        ],
        compiler_params=pltpu.CompilerParams(needs_layout_passes=False),
    )
    def body(in_hbm, bel_hbm, pl_hbm, inbuf, belbuf, plbuf):
        wid = lax.axis_index("s") * 2 + lax.axis_index("c")
        iota = lax.iota(jnp.int32, _L)
        pat = jnp.where(iota % 4 == 3, 1.0, 0.0).astype(jnp.float32)

        def fill(i, c):
            belbuf[pl.ds(i * _L, _L)] = pat
            plbuf[pl.ds(i * _L, _L)] = pat
            return c

        lax.fori_loop(0, (4 * P) // _L, fill, 0)

        def grp(i, c):
            i3 = iota * 3 + i * (3 * _L)
            x0 = plsc.load_gather(inbuf, [i3])
            x1 = plsc.load_gather(inbuf, [i3 + 1])
            x2 = plsc.load_gather(inbuf, [i3 + 2])
            o4 = iota * 4 + i * (4 * _L)
            plsc.store_scatter(belbuf, [o4 + 1], x0)
            plsc.store_scatter(belbuf, [o4 + 2], x1)
            plsc.store_scatter(plbuf, [o4 + 1], x0 + x2)
            plsc.store_scatter(plbuf, [o4 + 2], x1 + x2)
            return c

        def chunk(ci, c):
            base = (wid * CH + ci) * P
            pltpu.sync_copy(in_hbm.at[pl.ds(base * 3, 3 * P)], inbuf)
            lax.fori_loop(0, GRPS, grp, 0)
            pltpu.sync_copy(belbuf, bel_hbm.at[pl.ds(base * 4, 4 * P)])
            pltpu.sync_copy(plbuf, pl_hbm.at[pl.ds(base * 4, 4 * P)])
            return c

        lax.fori_loop(0, CH, chunk, 0)

    return body


def kernel(inputs):
    B, H, W, C = inputs.shape
    assert C == 3, "kernel specialized for a 2-class frame (3 input channels)"
    n_pix = B * H * W
    flat = inputs.reshape(n_pix * 3)
    bel_flat, pl_flat = _build(n_pix)(flat)
    return (bel_flat.reshape(B, H, W, 4), pl_flat.reshape(B, H, W, 4))


# speedup vs baseline: 105.5462x; 105.5462x over previous
import functools

import jax
import jax.numpy as jnp
from jax import lax
from jax.experimental import pallas as pl
from jax.experimental.pallas import tpu as pltpu
from jax.experimental.pallas import tpu_sc as plsc

_L = 16


@functools.lru_cache(maxsize=None)
def _build(B: int, W: int, H: int):
    NW = 32
    WC = 32
    units = (B * W) // WC
    assert units * WC == B * W and W % WC == 0
    max_units_per_w = (units + NW - 1) // NW
    w_per_b = W // WC
    HG = H // _L
    assert HG * _L == H

    mesh = plsc.VectorSubcoreMesh(core_axis_name="c", subcore_axis_name="s")

    @functools.partial(
        pl.kernel,
        mesh=mesh,
        out_type=(
            jax.ShapeDtypeStruct((B, W, 4, H), jnp.float32),
            jax.ShapeDtypeStruct((B, W, 4, H), jnp.float32),
        ),
        scratch_types=[
            pltpu.VMEM((WC, H), jnp.float32),
            pltpu.VMEM((WC, H), jnp.float32),
            pltpu.VMEM((WC, H), jnp.float32),
            pltpu.VMEM((WC, H), jnp.float32),
            pltpu.VMEM((WC, H), jnp.float32),
            pltpu.VMEM((WC, H), jnp.float32),
            pltpu.VMEM((WC, H), jnp.float32),
        ],
        compiler_params=pltpu.CompilerParams(needs_layout_passes=False),
    )
    def body(in_hbm, bel_hbm, pl_hbm, x0, x1, x2, p0, p1, zb, ob):
        wid = lax.axis_index("s") * 2 + lax.axis_index("c")
        zero_v = jnp.zeros((_L,), jnp.float32)
        one_v = jnp.ones((_L,), jnp.float32)

        def const_fill(i, c):
            w = i // HG
            h0 = (i % HG) * _L
            zb[w, pl.ds(h0, _L)] = zero_v
            ob[w, pl.ds(h0, _L)] = one_v
            return c

        lax.fori_loop(0, WC * HG, const_fill, 0)

        def add_body(i, c):
            w = i // HG
            hs = pl.ds((i % HG) * _L, _L)
            v2 = x2[w, hs]
            p0[w, hs] = x0[w, hs] + v2
            p1[w, hs] = x1[w, hs] + v2
            return c

        def do_unit(u):
            b = u // w_per_b
            w0 = (u % w_per_b) * WC
            ws = pl.ds(w0, WC)
            pltpu.sync_copy(in_hbm.at[b, 0, ws], x0)
            pltpu.sync_copy(in_hbm.at[b, 1, ws], x1)
            pltpu.sync_copy(in_hbm.at[b, 2, ws], x2)
            lax.fori_loop(0, WC * HG, add_body, 0)
            pltpu.sync_copy(zb, bel_hbm.at[b, ws, 0])
            pltpu.sync_copy(x0, bel_hbm.at[b, ws, 1])
            pltpu.sync_copy(x1, bel_hbm.at[b, ws, 2])
            pltpu.sync_copy(ob, bel_hbm.at[b, ws, 3])
            pltpu.sync_copy(zb, pl_hbm.at[b, ws, 0])
            pltpu.sync_copy(p0, pl_hbm.at[b, ws, 1])
            pltpu.sync_copy(p1, pl_hbm.at[b, ws, 2])
            pltpu.sync_copy(ob, pl_hbm.at[b, ws, 3])

        for k in range(max_units_per_w):
            u = k * NW + wid
            if (units % NW) == 0:
                do_unit(u)
            else:
                @pl.when(u < units)
                def _():
                    do_unit(u)

    return body


def kernel(inputs):
    B, H, W, C = inputs.shape
    assert C == 3, "kernel specialized for a 2-class frame (3 input channels)"
    xt = jnp.transpose(inputs, (0, 3, 2, 1))
    bel_t, pl_t = _build(B, W, H)(xt)
    bel = jnp.transpose(bel_t, (0, 3, 1, 2))
    pl_full = jnp.transpose(pl_t, (0, 3, 1, 2))
    return (bel, pl_full)


# speedup vs baseline: 142.5390x; 1.3505x over previous
import functools

import jax
import jax.numpy as jnp
from jax import lax
from jax.experimental import pallas as pl
from jax.experimental.pallas import tpu as pltpu
from jax.experimental.pallas import tpu_sc as plsc

_L = 16
_NSLOT = 3


@functools.lru_cache(maxsize=None)
def _build(B: int, W: int, H: int):
    NW = 32
    WC = 16
    units = (B * W) // WC
    assert units * WC == B * W and W % WC == 0
    upb = W // WC
    HG = H // _L
    assert HG * _L == H
    MAXK = (units + NW - 1) // NW
    full_k = units - (MAXK - 1) * NW

    mesh = plsc.VectorSubcoreMesh(core_axis_name="c", subcore_axis_name="s")

    data_bufs = [pltpu.VMEM((WC, H), jnp.float32) for _ in range(5 * _NSLOT)]
    const_bufs = [pltpu.VMEM((WC, H), jnp.float32) for _ in range(2)]
    sems = [pltpu.SemaphoreType.DMA for _ in range(2 * _NSLOT)]

    @functools.partial(
        pl.kernel,
        mesh=mesh,
        out_type=(
            jax.ShapeDtypeStruct((B, W, 4, H), jnp.float32),
            jax.ShapeDtypeStruct((B, W, 4, H), jnp.float32),
        ),
        scratch_types=data_bufs + const_bufs + sems,
        compiler_params=pltpu.CompilerParams(needs_layout_passes=False),
    )
    def body(in_hbm, bel_hbm, pl_hbm, *sc):
        bufs = [sc[5 * s:5 * s + 5] for s in range(_NSLOT)]
        zb, ob = sc[5 * _NSLOT], sc[5 * _NSLOT + 1]
        sin = sc[5 * _NSLOT + 2:5 * _NSLOT + 2 + _NSLOT]
        sout = sc[5 * _NSLOT + 2 + _NSLOT:]
        wid = lax.axis_index("s") * 2 + lax.axis_index("c")
        zero_v = jnp.zeros((_L,), jnp.float32)
        one_v = jnp.ones((_L,), jnp.float32)

        def const_fill(w, c):
            for hg in range(HG):
                hs = pl.ds(hg * _L, _L)
                zb[w, hs] = zero_v
                ob[w, hs] = one_v
            return c

        lax.fori_loop(0, WC, const_fill, 0)

        def unit_pos(k):
            u = k * NW + wid
            return u // upb, pl.ds((u % upb) * WC, WC)

        def in_copies(k):
            b, ws = unit_pos(k)
            x0s, x1s, x2s, _, _ = bufs[k % _NSLOT]
            sem = sin[k % _NSLOT]
            return [(in_hbm.at[b, 0, ws], x0s, sem),
                    (in_hbm.at[b, 1, ws], x1s, sem),
                    (in_hbm.at[b, 2, ws], x2s, sem)]

        def out_copies(k):
            b, ws = unit_pos(k)
            x0s, x1s, x2s, p0s, p1s = bufs[k % _NSLOT]
            sem = sout[k % _NSLOT]
            return [(zb, bel_hbm.at[b, ws, 0], sem),
                    (x0s, bel_hbm.at[b, ws, 1], sem),
                    (x1s, bel_hbm.at[b, ws, 2], sem),
                    (ob, bel_hbm.at[b, ws, 3], sem),
                    (zb, pl_hbm.at[b, ws, 0], sem),
                    (p0s, pl_hbm.at[b, ws, 1], sem),
                    (p1s, pl_hbm.at[b, ws, 2], sem),
                    (ob, pl_hbm.at[b, ws, 3], sem)]

        def issue(copies):
            for src, dst, sem in copies:
                pltpu.async_copy(src, dst, sem)

        def drain(copies):
            for src, dst, sem in copies:
                pltpu.make_async_copy(src, dst, sem).wait()

        def compute(k):
            x0s, x1s, x2s, p0s, p1s = bufs[k % _NSLOT]

            def wbody(w, c):
                for hg in range(HG):
                    hs = pl.ds(hg * _L, _L)
                    v2 = x2s[w, hs]
                    p0s[w, hs] = x0s[w, hs] + v2
                    p1s[w, hs] = x1s[w, hs] + v2
                return c

            lax.fori_loop(0, WC, wbody, 0)

        def guarded(k, fn):
            if k < MAXK - 1 or full_k == NW:
                fn()
            else:
                pl.when(wid < full_k)(fn)

        issue(in_copies(0))
        for k in range(MAXK):
            if k >= 2:
                drain(out_copies(k - 2))
            if k + 1 < MAXK:
                guarded(k + 1, lambda k=k: issue(in_copies(k + 1)))

            def stage(k=k):
                drain(in_copies(k))
                compute(k)
                issue(out_copies(k))

            guarded(k, stage)
        drain(out_copies(MAXK - 2))
        guarded(MAXK - 1, lambda: drain(out_copies(MAXK - 1)))

    return body


def kernel(inputs):
    B, H, W, C = inputs.shape
    assert C == 3, "kernel specialized for a 2-class frame (3 input channels)"
    xt = jnp.transpose(inputs, (0, 3, 2, 1))
    bel_t, pl_t = _build(B, W, H)(xt)
    bel = jnp.transpose(bel_t, (0, 3, 1, 2))
    pl_full = jnp.transpose(pl_t, (0, 3, 1, 2))
    return (bel, pl_full)
